# baseline (speedup 1.0000x reference)
import jax
import jax.numpy as jnp
from jax.experimental import pallas as pl
from jax.experimental.pallas import tpu as pltpu

BLK_S = 512


def _add_kernel(idx_ref, x_ref, emb_ref, o_ref):
    o_ref[...] = x_ref[...] + emb_ref[...]


def kernel(x, tissue_vector, registry_tokens):
    B, S, D = x.shape
    idx = tissue_vector[:, 0].astype(jnp.int32)
    table = registry_tokens.reshape(registry_tokens.shape[0], 1, D)
    grid = (B, S // BLK_S)
    out = pl.pallas_call(
        _add_kernel,
        grid_spec=pltpu.PrefetchScalarGridSpec(
            num_scalar_prefetch=1,
            grid=grid,
            in_specs=[
                pl.BlockSpec((1, BLK_S, D), lambda b, s, idx_ref: (b, s, 0)),
                pl.BlockSpec((1, 1, D), lambda b, s, idx_ref: (idx_ref[b], 0, 0)),
            ],
            out_specs=pl.BlockSpec((1, BLK_S, D), lambda b, s, idx_ref: (b, s, 0)),
        ),
        out_shape=jax.ShapeDtypeStruct((B, S, D), x.dtype),
    )(idx, x, table)
    return out


# speedup vs baseline: 1.0171x; 1.0171x over previous
import jax
import jax.numpy as jnp
from jax.experimental import pallas as pl
from jax.experimental.pallas import tpu as pltpu

BLK_S = 1024


def _add_kernel(idx_ref, x_ref, emb_ref, o_ref):
    o_ref[...] = x_ref[...] + emb_ref[...]


def kernel(x, tissue_vector, registry_tokens):
    B, S, D = x.shape
    idx = tissue_vector[:, 0].astype(jnp.int32)
    table = registry_tokens.reshape(registry_tokens.shape[0], 1, D)
    grid = (B, S // BLK_S)
    out = pl.pallas_call(
        _add_kernel,
        grid_spec=pltpu.PrefetchScalarGridSpec(
            num_scalar_prefetch=1,
            grid=grid,
            in_specs=[
                pl.BlockSpec((1, BLK_S, D), lambda b, s, idx_ref: (b, s, 0)),
                pl.BlockSpec((1, 1, D), lambda b, s, idx_ref: (idx_ref[b], 0, 0)),
            ],
            out_specs=pl.BlockSpec((1, BLK_S, D), lambda b, s, idx_ref: (b, s, 0)),
        ),
        out_shape=jax.ShapeDtypeStruct((B, S, D), x.dtype),
        compiler_params=pltpu.CompilerParams(
            dimension_semantics=("parallel", "parallel"),
        ),
    )(idx, x, table)
    return out
